# baseline (speedup 1.0000x reference)
import functools

import jax
import jax.numpy as jnp
from jax import lax
from jax.experimental import pallas as pl
from jax.experimental.pallas import tpu as pltpu
from jax.experimental.pallas import tpu_sc as plsc

N_NODES = 10000
N_EDGES = 160000
IN_DIM = 256
H = 512
NHEAD = 8
HD = H // NHEAD
NUM_LAYERS = 2
T_STEPS = 1000
ALPHA_L = 2.0

HCOLS = HD + 16
NTILES = 16
EPT = 10112
EB = 128
NBLK = EPT // EB
N_ES = 10008
N_ACC = 10112
DUMMY = N_NODES
NEG = -1.0e30


def _mm_bias_body(x_ref, w_ref, b_ref, o_ref):
    o_ref[...] = (
        jnp.dot(x_ref[...], w_ref[...], preferred_element_type=jnp.float32)
        + b_ref[...]
    )


def _mm_bias(x, w, b, bm=2000):
    m, k = x.shape
    n = w.shape[1]
    if m <= bm:
        bm = m
    assert m % bm == 0
    return pl.pallas_call(
        _mm_bias_body,
        grid=(m // bm,),
        in_specs=[
            pl.BlockSpec((bm, k), lambda i: (i, 0)),
            pl.BlockSpec((k, n), lambda i: (0, 0)),
            pl.BlockSpec((1, n), lambda i: (0, 0)),
        ],
        out_specs=pl.BlockSpec((bm, n), lambda i: (i, 0)),
        out_shape=jax.ShapeDtypeStruct((m, n), jnp.float32),
    )(x, w, b.reshape(1, n))


def _onehot_body(t_ref, tab_ref, o_ref):
    tb = t_ref[...]
    oh = (lax.broadcasted_iota(jnp.int32, (tb.shape[0], T_STEPS), 1)
          == tb).astype(jnp.float32)
    o_ref[...] = jnp.dot(oh, tab_ref[...], preferred_element_type=jnp.float32)


def _onehot_gather(t, table, bm=1000):
    m = t.shape[0]
    n = table.shape[1]
    return pl.pallas_call(
        _onehot_body,
        grid=(m // bm,),
        in_specs=[
            pl.BlockSpec((bm, 1), lambda i: (i, 0)),
            pl.BlockSpec((T_STEPS, n), lambda i: (0, 0)),
        ],
        out_specs=pl.BlockSpec((bm, n), lambda i: (i, 0)),
        out_shape=jax.ShapeDtypeStruct((m, n), jnp.float32),
    )(t.reshape(m, 1), table)


def _hw_body(h_ref, w_ref, asrc_ref, adst_ref, hw_ref, es_ref, ed_ref):
    blk = jnp.dot(h_ref[...], w_ref[0], preferred_element_type=jnp.float32)
    bm = blk.shape[0]
    ii = lax.broadcasted_iota(jnp.int32, (bm, 16), 1)
    ones_pat = jnp.where(ii == 0, 1.0, 0.0).astype(jnp.float32)
    hw_ref[0] = jnp.concatenate([blk, ones_pat], axis=1)
    es_ref[0] = jnp.sum(blk * asrc_ref[0], axis=1, keepdims=True)
    ed_ref[0] = jnp.sum(blk * adst_ref[0], axis=1, keepdims=True)


def _hw_proj(h, W, a_src, a_dst, bm=2000):
    m = h.shape[0]
    grid = (m // bm, NHEAD)
    hw, es, ed = pl.pallas_call(
        _hw_body,
        grid=grid,
        in_specs=[
            pl.BlockSpec((bm, H), lambda i, j: (i, 0)),
            pl.BlockSpec((1, H, HD), lambda i, j: (j, 0, 0)),
            pl.BlockSpec((1, 1, HD), lambda i, j: (j, 0, 0)),
            pl.BlockSpec((1, 1, HD), lambda i, j: (j, 0, 0)),
        ],
        out_specs=[
            pl.BlockSpec((1, bm, HCOLS), lambda i, j: (j, i, 0)),
            pl.BlockSpec((1, bm, 1), lambda i, j: (j, i, 0)),
            pl.BlockSpec((1, bm, 1), lambda i, j: (j, i, 0)),
        ],
        out_shape=[
            jax.ShapeDtypeStruct((NHEAD, m, HCOLS), jnp.float32),
            jax.ShapeDtypeStruct((NHEAD, m, 1), jnp.float32),
            jax.ShapeDtypeStruct((NHEAD, m, 1), jnp.float32),
        ],
    )(h, W.reshape(H, NHEAD, HD).transpose(1, 0, 2),
      a_src.reshape(NHEAD, 1, HD), a_dst.reshape(NHEAD, 1, HD))
    return hw, es, ed


def _edge_body(hwp, esp, edp, src1p, dst2p, nump,
               src1_v, srcoff_v, dst2_v, es_v, ed_v,
               rows_v, ee_v, num_s, sem, sem_sc):
    c = lax.axis_index("c")
    s = lax.axis_index("s")
    zf16 = jnp.zeros((16,), jnp.float32)

    pltpu.sync_copy(src1p.at[pl.ds(s * EPT, EPT)], src1_v)
    pltpu.sync_copy(dst2p.at[s], dst2_v)

    for phase in range(4):
        head = c * 4 + phase

        pltpu.sync_copy(esp.at[pl.ds(head * N_ES, N_ES)], es_v)
        pltpu.sync_copy(edp.at[pl.ds(head * N_ES, N_ES)], ed_v)

        def _off(k, _):
            srcoff_v[pl.ds(k * 16, 16)] = (
                src1_v[pl.ds(k * 16, 16)] + head * N_NODES)
            return 0
        lax.fori_loop(0, EPT // 16, _off, 0)

        def _zr(r, _):
            for j in range(HCOLS // 16):
                rows_v[r, pl.ds(j * 16, 16)] = zf16
            return 0
        lax.fori_loop(0, EB, _zr, 0)
        base = s * (N_ACC // NTILES)
        for k in range(4):
            pltpu.sync_copy(rows_v.at[pl.ds(0, EB)], num_s.at[pl.ds(base + k * EB, EB)])
        pltpu.sync_copy(rows_v.at[pl.ds(0, 120)],
                        num_s.at[pl.ds(base + 4 * EB, 120)])
        plsc.subcore_barrier()

        def _gather(b, par):
            return pltpu.make_async_copy(
                hwp.at[srcoff_v.at[pl.ds(b * EB, EB)]],
                rows_v.at[pl.ds(par * EB, EB)], sem)

        def _scatter(b, par):
            return pltpu.make_async_copy(
                rows_v.at[pl.ds(par * EB, EB)],
                num_s.at[dst2_v.at[b]], sem_sc)

        _gather(0, 0).start()
        def _blk(b, _):
            par = lax.rem(b, 2)
            @pl.when(b >= 1)
            def _drain():
                _scatter(b - 1, 1 - par).wait()
            @pl.when(b < NBLK - 1)
            def _next():
                _gather(b + 1, 1 - par).start()
            _gather(b, par).wait()
            ro = par * EB
            for j in range(8):
                s16 = src1_v[pl.ds(b * EB + j * 16, 16)]
                d16 = dst2_v[b, pl.ds(j * 16, 16)]
                ev = (plsc.load_gather(es_v, [s16])
                      + plsc.load_gather(ed_v, [d16]))
                ev = jnp.where(ev >= 0.0, ev, 0.2 * ev)
                ee_v[pl.ds(j * 16, 16)] = jnp.exp(ev)
            def _scale(e2, _):
                for u in range(2):
                    e = e2 * 2 + u
                    e16 = jnp.full((16,), e, jnp.int32)
                    sc = plsc.load_gather(ee_v, [e16])
                    for k in range(HCOLS // 16):
                        o = k * 16
                        rows_v[ro + e, pl.ds(o, 16)] = (
                            rows_v[ro + e, pl.ds(o, 16)] * sc)
                return 0
            lax.fori_loop(0, EB // 2, _scale, 0)
            _scatter(b, par).start(add=True)
            return 0
        lax.fori_loop(0, NBLK, _blk, 0)
        _scatter(NBLK - 1, (NBLK - 1) % 2).wait()
        plsc.subcore_barrier()

        rb = s * 624
        pltpu.sync_copy(num_s.at[pl.ds(rb, 624)],
                        nump.at[pl.ds(head * N_NODES + rb, 624)])
        @pl.when(s == NTILES - 1)
        def _tail():
            pltpu.sync_copy(num_s.at[pl.ds(9984, 16)],
                            nump.at[pl.ds(head * N_NODES + 9984, 16)])
        plsc.subcore_barrier()


_edge_kernel = pl.kernel(
    _edge_body,
    out_type=jax.ShapeDtypeStruct((NHEAD * N_NODES, HCOLS), jnp.float32),
    mesh=plsc.VectorSubcoreMesh(core_axis_name="c", subcore_axis_name="s"),
    compiler_params=pltpu.CompilerParams(
        needs_layout_passes=False, use_tc_tiling_on_sc=False),
    scratch_types=[
        pltpu.VMEM((EPT,), jnp.int32),
        pltpu.VMEM((EPT,), jnp.int32),
        pltpu.VMEM((NBLK, EB), jnp.int32),
        pltpu.VMEM((N_ES,), jnp.float32),
        pltpu.VMEM((N_ES,), jnp.float32),
        pltpu.VMEM((2 * EB, HCOLS), jnp.float32),
        pltpu.VMEM((EB,), jnp.float32),
        pltpu.VMEM_SHARED((N_ACC, HCOLS), jnp.float32),
        pltpu.SemaphoreType.DMA,
        pltpu.SemaphoreType.DMA,
    ],
)


def _gat_sc(h, p, src1p, dst2p):
    hw, es, ed = _hw_proj(h, p['W'], p['a_src'], p['a_dst'])
    pad = ((0, 0), (0, N_ES - N_NODES), (0, 0))
    esp = jnp.pad(es, pad, constant_values=NEG).reshape(NHEAD * N_ES)
    edp = jnp.pad(ed, pad, constant_values=NEG).reshape(NHEAD * N_ES)
    nump = _edge_kernel(
        hw.reshape(NHEAD * N_NODES, HCOLS), esp, edp, src1p, dst2p)
    acc = nump.reshape(NHEAD, N_NODES, HCOLS).transpose(1, 0, 2)
    num = acc[:, :, :HD]
    den = acc[:, :, HD:HD + 1]
    out = num / (den + 1e-9)
    return out.reshape(N_NODES, H)


def _ln(x, scale=None, bias=None, eps=1e-5):
    mu = jnp.mean(x, axis=-1, keepdims=True)
    var = jnp.var(x, axis=-1, keepdims=True)
    y = (x - mu) / jnp.sqrt(var + eps)
    if scale is not None:
        y = y * scale + bias
    return y


def kernel(x, edge_index, t, noise_raw, params):
    src, dst = edge_index[0], edge_index[1]
    npad = NTILES * EPT - N_EDGES
    src1p = jnp.concatenate([src, jnp.zeros((npad,), jnp.int32)])
    dstf = jnp.concatenate([dst, jnp.full((npad,), DUMMY, jnp.int32)])
    dst2p = dstf.reshape(NTILES, NBLK, EB)

    betas = jnp.linspace(1e-4, 0.02, T_STEPS, dtype=jnp.float32)
    alphas_bar = jnp.cumprod(1.0 - betas)
    sab = jnp.sqrt(alphas_bar)
    somab = jnp.sqrt(1.0 - alphas_bar)

    layers = list(params['down']) + list(params['up'])
    tabs = [_mm_bias(params['time_embedding'], p['tW'], p['tb'])
            for p in layers]
    table = jnp.concatenate(
        tabs + [sab.reshape(T_STEPS, 1), somab.reshape(T_STEPS, 1),
                jnp.zeros((T_STEPS, 126), jnp.float32)], axis=1)
    g = _onehot_gather(t, table)
    temb_proj = [g[:, i * H:(i + 1) * H] for i in range(4)]
    sab_t = g[:, 4 * H:4 * H + 1]
    somab_t = g[:, 4 * H + 1:4 * H + 2]

    xn = _ln(x)
    miu = jnp.mean(xn, axis=0)
    std = jnp.std(xn, axis=0, ddof=1)
    noise = _ln(noise_raw)
    noise = noise * std + miu
    noise = jnp.sign(xn) * jnp.abs(noise)
    x_t = sab_t * xn + somab_t * noise

    h = _mm_bias(x_t, params['W_in'], params['b_in'])
    skips = []
    for i in range(NUM_LAYERS):
        p = params['down'][i]
        h = h + temb_proj[i]
        h = _gat_sc(h, p, src1p, dst2p)
        y = _ln(h, p['ln_s'], p['ln_b'])
        h = jnp.where(y >= 0, y, p['prelu'] * y)
        skips.append(h)
    for i in range(NUM_LAYERS):
        p = params['up'][i]
        h = h + skips[NUM_LAYERS - 1 - i]
        h = h + temb_proj[NUM_LAYERS + i]
        h = _gat_sc(h, p, src1p, dst2p)
        y = _ln(h, p['ln_s'], p['ln_b'])
        h = jnp.where(y >= 0, y, p['prelu'] * y)
    pred = _mm_bias(h, params['W_out'], params['b_out'])

    pn = pred / (jnp.linalg.norm(pred, axis=-1, keepdims=True) + 1e-12)
    tn = xn / (jnp.linalg.norm(xn, axis=-1, keepdims=True) + 1e-12)
    return jnp.mean((1.0 - jnp.sum(pn * tn, axis=-1)) ** ALPHA_L)


# speedup vs baseline: 1.0071x; 1.0071x over previous
import functools

import jax
import jax.numpy as jnp
from jax import lax
from jax.experimental import pallas as pl
from jax.experimental.pallas import tpu as pltpu
from jax.experimental.pallas import tpu_sc as plsc

N_NODES = 10000
N_EDGES = 160000
IN_DIM = 256
H = 512
NHEAD = 8
HD = H // NHEAD
NUM_LAYERS = 2
T_STEPS = 1000
ALPHA_L = 2.0

HCOLS = HD + 16
NTILES = 16
EPT = 10112
EB = 128
NBLK = EPT // EB
N_ES = 10008
N_ACC = 10112
DUMMY = N_NODES
NEG = -1.0e30


def _mm_bias_body(x_ref, w_ref, b_ref, o_ref):
    o_ref[...] = (
        jnp.dot(x_ref[...], w_ref[...], preferred_element_type=jnp.float32)
        + b_ref[...]
    )


def _mm_bias(x, w, b, bm=2000):
    m, k = x.shape
    n = w.shape[1]
    if m <= bm:
        bm = m
    assert m % bm == 0
    return pl.pallas_call(
        _mm_bias_body,
        grid=(m // bm,),
        in_specs=[
            pl.BlockSpec((bm, k), lambda i: (i, 0)),
            pl.BlockSpec((k, n), lambda i: (0, 0)),
            pl.BlockSpec((1, n), lambda i: (0, 0)),
        ],
        out_specs=pl.BlockSpec((bm, n), lambda i: (i, 0)),
        out_shape=jax.ShapeDtypeStruct((m, n), jnp.float32),
    )(x, w, b.reshape(1, n))


def _onehot_body(t_ref, tab_ref, o_ref):
    tb = t_ref[...]
    oh = (lax.broadcasted_iota(jnp.int32, (tb.shape[0], T_STEPS), 1)
          == tb).astype(jnp.float32)
    o_ref[...] = jnp.dot(oh, tab_ref[...], preferred_element_type=jnp.float32)


def _onehot_gather(t, table, bm=1000):
    m = t.shape[0]
    n = table.shape[1]
    return pl.pallas_call(
        _onehot_body,
        grid=(m // bm,),
        in_specs=[
            pl.BlockSpec((bm, 1), lambda i: (i, 0)),
            pl.BlockSpec((T_STEPS, n), lambda i: (0, 0)),
        ],
        out_specs=pl.BlockSpec((bm, n), lambda i: (i, 0)),
        out_shape=jax.ShapeDtypeStruct((m, n), jnp.float32),
    )(t.reshape(m, 1), table)


def _hw_body(h_ref, w_ref, asrc_ref, adst_ref, hw_ref, es_ref, ed_ref):
    blk = jnp.dot(h_ref[...], w_ref[0], preferred_element_type=jnp.float32)
    bm = blk.shape[0]
    ii = lax.broadcasted_iota(jnp.int32, (bm, 16), 1)
    ones_pat = jnp.where(ii == 0, 1.0, 0.0).astype(jnp.float32)
    hw_ref[0] = jnp.concatenate([blk, ones_pat], axis=1)
    es_ref[0] = jnp.sum(blk * asrc_ref[0], axis=1, keepdims=True)
    ed_ref[0] = jnp.sum(blk * adst_ref[0], axis=1, keepdims=True)


def _hw_proj(h, W, a_src, a_dst, bm=2000):
    m = h.shape[0]
    grid = (m // bm, NHEAD)
    hw, es, ed = pl.pallas_call(
        _hw_body,
        grid=grid,
        in_specs=[
            pl.BlockSpec((bm, H), lambda i, j: (i, 0)),
            pl.BlockSpec((1, H, HD), lambda i, j: (j, 0, 0)),
            pl.BlockSpec((1, 1, HD), lambda i, j: (j, 0, 0)),
            pl.BlockSpec((1, 1, HD), lambda i, j: (j, 0, 0)),
        ],
        out_specs=[
            pl.BlockSpec((1, bm, HCOLS), lambda i, j: (j, i, 0)),
            pl.BlockSpec((1, bm, 1), lambda i, j: (j, i, 0)),
            pl.BlockSpec((1, bm, 1), lambda i, j: (j, i, 0)),
        ],
        out_shape=[
            jax.ShapeDtypeStruct((NHEAD, m, HCOLS), jnp.float32),
            jax.ShapeDtypeStruct((NHEAD, m, 1), jnp.float32),
            jax.ShapeDtypeStruct((NHEAD, m, 1), jnp.float32),
        ],
    )(h, W.reshape(H, NHEAD, HD).transpose(1, 0, 2),
      a_src.reshape(NHEAD, 1, HD), a_dst.reshape(NHEAD, 1, HD))
    return hw, es, ed


def _edge_body(hwp, esp, edp, src1p, dst2p, nump,
               src1_v, srcoff_v, dst2_v, es_v, ed_v,
               rows_v, ee_v, num_s, sem, sem_sc):
    c = lax.axis_index("c")
    s = lax.axis_index("s")
    zf16 = jnp.zeros((16,), jnp.float32)

    pltpu.sync_copy(src1p.at[pl.ds(s * EPT, EPT)], src1_v)
    pltpu.sync_copy(dst2p.at[s], dst2_v)

    for phase in range(4):
        head = c * 4 + phase

        pltpu.sync_copy(esp.at[pl.ds(head * N_ES, N_ES)], es_v)
        pltpu.sync_copy(edp.at[pl.ds(head * N_ES, N_ES)], ed_v)

        def _off(k, _):
            srcoff_v[pl.ds(k * 16, 16)] = (
                src1_v[pl.ds(k * 16, 16)] + head * N_NODES)
            return 0
        lax.fori_loop(0, EPT // 16, _off, 0)

        def _zr(r, _):
            for j in range(HCOLS // 16):
                rows_v[r, pl.ds(j * 16, 16)] = zf16
            return 0
        lax.fori_loop(0, EB, _zr, 0)
        base = s * (N_ACC // NTILES)
        for k in range(4):
            pltpu.sync_copy(rows_v.at[pl.ds(0, EB)], num_s.at[pl.ds(base + k * EB, EB)])
        pltpu.sync_copy(rows_v.at[pl.ds(0, 120)],
                        num_s.at[pl.ds(base + 4 * EB, 120)])
        plsc.subcore_barrier()

        def _gather(b, par):
            return pltpu.make_async_copy(
                hwp.at[srcoff_v.at[pl.ds(b * EB, EB)]],
                rows_v.at[pl.ds(par * EB, EB)], sem)

        def _scatter(b, par):
            return pltpu.make_async_copy(
                rows_v.at[pl.ds(par * EB, EB)],
                num_s.at[dst2_v.at[b]], sem_sc)

        _gather(0, 0).start()
        def _blk(b, _):
            par = lax.rem(b, 2)
            @pl.when(b >= 1)
            def _drain():
                _scatter(b - 1, 1 - par).wait()
            @pl.when(b < NBLK - 1)
            def _next():
                _gather(b + 1, 1 - par).start()
            _gather(b, par).wait()
            ro = par * EB
            for j in range(8):
                s16 = src1_v[pl.ds(b * EB + j * 16, 16)]
                d16 = dst2_v[b, pl.ds(j * 16, 16)]
                ev = (plsc.load_gather(es_v, [s16])
                      + plsc.load_gather(ed_v, [d16]))
                ev = jnp.where(ev >= 0.0, ev, 0.2 * ev)
                ee_v[pl.ds(j * 16, 16)] = jnp.exp(ev)
            def _scale(e, _):
                e16 = jnp.full((16,), e, jnp.int32)
                sc = plsc.load_gather(ee_v, [e16])
                for k in range(HCOLS // 16):
                    o = k * 16
                    rows_v[ro + e, pl.ds(o, 16)] = (
                        rows_v[ro + e, pl.ds(o, 16)] * sc)
                return 0
            lax.fori_loop(0, EB, _scale, 0)
            _scatter(b, par).start(add=True)
            return 0
        lax.fori_loop(0, NBLK, _blk, 0)
        _scatter(NBLK - 1, (NBLK - 1) % 2).wait()
        plsc.subcore_barrier()

        rb = s * 624
        pltpu.sync_copy(num_s.at[pl.ds(rb, 624)],
                        nump.at[pl.ds(head * N_NODES + rb, 624)])
        @pl.when(s == NTILES - 1)
        def _tail():
            pltpu.sync_copy(num_s.at[pl.ds(9984, 16)],
                            nump.at[pl.ds(head * N_NODES + 9984, 16)])
        plsc.subcore_barrier()


_edge_kernel = pl.kernel(
    _edge_body,
    out_type=jax.ShapeDtypeStruct((NHEAD * N_NODES, HCOLS), jnp.float32),
    mesh=plsc.VectorSubcoreMesh(core_axis_name="c", subcore_axis_name="s"),
    compiler_params=pltpu.CompilerParams(
        needs_layout_passes=False, use_tc_tiling_on_sc=False),
    scratch_types=[
        pltpu.VMEM((EPT,), jnp.int32),
        pltpu.VMEM((EPT,), jnp.int32),
        pltpu.VMEM((NBLK, EB), jnp.int32),
        pltpu.VMEM((N_ES,), jnp.float32),
        pltpu.VMEM((N_ES,), jnp.float32),
        pltpu.VMEM((2 * EB, HCOLS), jnp.float32),
        pltpu.VMEM((EB,), jnp.float32),
        pltpu.VMEM_SHARED((N_ACC, HCOLS), jnp.float32),
        pltpu.SemaphoreType.DMA,
        pltpu.SemaphoreType.DMA,
    ],
)


def _gat_sc(h, p, src1p, dst2p):
    hw, es, ed = _hw_proj(h, p['W'], p['a_src'], p['a_dst'])
    pad = ((0, 0), (0, N_ES - N_NODES), (0, 0))
    esp = jnp.pad(es, pad, constant_values=NEG).reshape(NHEAD * N_ES)
    edp = jnp.pad(ed, pad, constant_values=NEG).reshape(NHEAD * N_ES)
    nump = _edge_kernel(
        hw.reshape(NHEAD * N_NODES, HCOLS), esp, edp, src1p, dst2p)
    acc = nump.reshape(NHEAD, N_NODES, HCOLS).transpose(1, 0, 2)
    num = acc[:, :, :HD]
    den = acc[:, :, HD:HD + 1]
    out = num / (den + 1e-9)
    return out.reshape(N_NODES, H)


def _ln(x, scale=None, bias=None, eps=1e-5):
    mu = jnp.mean(x, axis=-1, keepdims=True)
    var = jnp.var(x, axis=-1, keepdims=True)
    y = (x - mu) / jnp.sqrt(var + eps)
    if scale is not None:
        y = y * scale + bias
    return y


def kernel(x, edge_index, t, noise_raw, params):
    src, dst = edge_index[0], edge_index[1]
    npad = NTILES * EPT - N_EDGES
    src1p = jnp.concatenate([src, jnp.zeros((npad,), jnp.int32)])
    dstf = jnp.concatenate([dst, jnp.full((npad,), DUMMY, jnp.int32)])
    dst2p = dstf.reshape(NTILES, NBLK, EB)

    betas = jnp.linspace(1e-4, 0.02, T_STEPS, dtype=jnp.float32)
    alphas_bar = jnp.cumprod(1.0 - betas)
    sab = jnp.sqrt(alphas_bar)
    somab = jnp.sqrt(1.0 - alphas_bar)

    layers = list(params['down']) + list(params['up'])
    tabs = [_mm_bias(params['time_embedding'], p['tW'], p['tb'])
            for p in layers]
    table = jnp.concatenate(
        tabs + [sab.reshape(T_STEPS, 1), somab.reshape(T_STEPS, 1),
                jnp.zeros((T_STEPS, 126), jnp.float32)], axis=1)
    g = _onehot_gather(t, table)
    temb_proj = [g[:, i * H:(i + 1) * H] for i in range(4)]
    sab_t = g[:, 4 * H:4 * H + 1]
    somab_t = g[:, 4 * H + 1:4 * H + 2]

    xn = _ln(x)
    miu = jnp.mean(xn, axis=0)
    std = jnp.std(xn, axis=0, ddof=1)
    noise = _ln(noise_raw)
    noise = noise * std + miu
    noise = jnp.sign(xn) * jnp.abs(noise)
    x_t = sab_t * xn + somab_t * noise

    h = _mm_bias(x_t, params['W_in'], params['b_in'])
    skips = []
    for i in range(NUM_LAYERS):
        p = params['down'][i]
        h = h + temb_proj[i]
        h = _gat_sc(h, p, src1p, dst2p)
        y = _ln(h, p['ln_s'], p['ln_b'])
        h = jnp.where(y >= 0, y, p['prelu'] * y)
        skips.append(h)
    for i in range(NUM_LAYERS):
        p = params['up'][i]
        h = h + skips[NUM_LAYERS - 1 - i]
        h = h + temb_proj[NUM_LAYERS + i]
        h = _gat_sc(h, p, src1p, dst2p)
        y = _ln(h, p['ln_s'], p['ln_b'])
        h = jnp.where(y >= 0, y, p['prelu'] * y)
    pred = _mm_bias(h, params['W_out'], params['b_out'])

    pn = pred / (jnp.linalg.norm(pred, axis=-1, keepdims=True) + 1e-12)
    tn = xn / (jnp.linalg.norm(xn, axis=-1, keepdims=True) + 1e-12)
    return jnp.mean((1.0 - jnp.sum(pn * tn, axis=-1)) ** ALPHA_L)


# speedup vs baseline: 1.0594x; 1.0520x over previous
import functools

import jax
import jax.numpy as jnp
from jax import lax
from jax.experimental import pallas as pl
from jax.experimental.pallas import tpu as pltpu
from jax.experimental.pallas import tpu_sc as plsc

N_NODES = 10000
N_EDGES = 160000
IN_DIM = 256
H = 512
NHEAD = 8
HD = H // NHEAD
NUM_LAYERS = 2
T_STEPS = 1000
ALPHA_L = 2.0

HCOLS = HD + 16
NTILES = 16
EPT = 10112
EB = 128
NBLK = EPT // EB
N_ES = 10008
N_ACC = 10112
DUMMY = N_NODES
NEG = -1.0e30


def _mm_bias_body(x_ref, w_ref, b_ref, o_ref):
    o_ref[...] = (
        jnp.dot(x_ref[...], w_ref[...], preferred_element_type=jnp.float32)
        + b_ref[...]
    )


def _mm_bias(x, w, b, bm=2000):
    m, k = x.shape
    n = w.shape[1]
    if m <= bm:
        bm = m
    assert m % bm == 0
    return pl.pallas_call(
        _mm_bias_body,
        grid=(m // bm,),
        in_specs=[
            pl.BlockSpec((bm, k), lambda i: (i, 0)),
            pl.BlockSpec((k, n), lambda i: (0, 0)),
            pl.BlockSpec((1, n), lambda i: (0, 0)),
        ],
        out_specs=pl.BlockSpec((bm, n), lambda i: (i, 0)),
        out_shape=jax.ShapeDtypeStruct((m, n), jnp.float32),
    )(x, w, b.reshape(1, n))


def _onehot_body(t_ref, tab_ref, o_ref):
    tb = t_ref[...]
    oh = (lax.broadcasted_iota(jnp.int32, (tb.shape[0], T_STEPS), 1)
          == tb).astype(jnp.float32)
    o_ref[...] = jnp.dot(oh, tab_ref[...], preferred_element_type=jnp.float32)


def _onehot_gather(t, table, bm=1000):
    m = t.shape[0]
    n = table.shape[1]
    return pl.pallas_call(
        _onehot_body,
        grid=(m // bm,),
        in_specs=[
            pl.BlockSpec((bm, 1), lambda i: (i, 0)),
            pl.BlockSpec((T_STEPS, n), lambda i: (0, 0)),
        ],
        out_specs=pl.BlockSpec((bm, n), lambda i: (i, 0)),
        out_shape=jax.ShapeDtypeStruct((m, n), jnp.float32),
    )(t.reshape(m, 1), table)


def _hw_body(h_ref, w_ref, asrc_ref, adst_ref, hw_ref, es_ref, ed_ref):
    blk = jnp.dot(h_ref[...], w_ref[0], preferred_element_type=jnp.float32)
    bm = blk.shape[0]
    ii = lax.broadcasted_iota(jnp.int32, (bm, 16), 1)
    ones_pat = jnp.where(ii == 0, 1.0, 0.0).astype(jnp.float32)
    hw_ref[0] = jnp.concatenate([blk, ones_pat], axis=1)
    es_ref[0] = jnp.sum(blk * asrc_ref[0], axis=1, keepdims=True)
    ed_ref[0] = jnp.sum(blk * adst_ref[0], axis=1, keepdims=True)


def _hw_proj(h, W, a_src, a_dst, bm=2000):
    m = h.shape[0]
    grid = (m // bm, NHEAD)
    hw, es, ed = pl.pallas_call(
        _hw_body,
        grid=grid,
        in_specs=[
            pl.BlockSpec((bm, H), lambda i, j: (i, 0)),
            pl.BlockSpec((1, H, HD), lambda i, j: (j, 0, 0)),
            pl.BlockSpec((1, 1, HD), lambda i, j: (j, 0, 0)),
            pl.BlockSpec((1, 1, HD), lambda i, j: (j, 0, 0)),
        ],
        out_specs=[
            pl.BlockSpec((1, bm, HCOLS), lambda i, j: (j, i, 0)),
            pl.BlockSpec((1, bm, 1), lambda i, j: (j, i, 0)),
            pl.BlockSpec((1, bm, 1), lambda i, j: (j, i, 0)),
        ],
        out_shape=[
            jax.ShapeDtypeStruct((NHEAD, m, HCOLS), jnp.float32),
            jax.ShapeDtypeStruct((NHEAD, m, 1), jnp.float32),
            jax.ShapeDtypeStruct((NHEAD, m, 1), jnp.float32),
        ],
    )(h, W.reshape(H, NHEAD, HD).transpose(1, 0, 2),
      a_src.reshape(NHEAD, 1, HD), a_dst.reshape(NHEAD, 1, HD))
    return hw, es, ed


def _edge_body(hwp, esp, edp, src1p, dst2p, nump,
               src1_v, srcoff_v, dst2_v, es_v, ed_v,
               rows_v, ee_v, num_s, sem, sem_sc):
    c = lax.axis_index("c")
    s = lax.axis_index("s")
    zf16 = jnp.zeros((16,), jnp.float32)

    pltpu.sync_copy(src1p.at[pl.ds(s * EPT, EPT)], src1_v)
    pltpu.sync_copy(dst2p.at[s], dst2_v)

    for phase in range(4):
        head = c * 4 + phase

        pltpu.sync_copy(esp.at[pl.ds(head * N_ES, N_ES)], es_v)
        pltpu.sync_copy(edp.at[pl.ds(head * N_ES, N_ES)], ed_v)

        def _off(k, _):
            srcoff_v[pl.ds(k * 16, 16)] = (
                src1_v[pl.ds(k * 16, 16)] + head * N_NODES)
            return 0
        lax.fori_loop(0, EPT // 16, _off, 0)

        def _zr(r, _):
            for j in range(HCOLS // 16):
                rows_v[r, pl.ds(j * 16, 16)] = zf16
            return 0
        lax.fori_loop(0, EB, _zr, 0)
        base = s * (N_ACC // NTILES)
        for k in range(4):
            pltpu.sync_copy(rows_v.at[pl.ds(0, EB)], num_s.at[pl.ds(base + k * EB, EB)])
        pltpu.sync_copy(rows_v.at[pl.ds(0, 120)],
                        num_s.at[pl.ds(base + 4 * EB, 120)])
        plsc.subcore_barrier()

        def _gather(b, par):
            return pltpu.make_async_copy(
                hwp.at[srcoff_v.at[pl.ds(b * EB, EB)]],
                rows_v.at[pl.ds(par * EB, EB)], sem)

        def _scatter(b, par):
            return pltpu.make_async_copy(
                rows_v.at[pl.ds(par * EB, EB)],
                num_s.at[dst2_v.at[b]], sem_sc)

        _gather(0, 0).start()
        def _blk(b, _):
            par = lax.rem(b, 2)
            @pl.when(b >= 1)
            def _drain():
                _scatter(b - 1, 1 - par).wait()
            @pl.when(b < NBLK - 1)
            def _next():
                _gather(b + 1, 1 - par).start()
            _gather(b, par).wait()
            ro = par * EB
            for j in range(8):
                s16 = src1_v[pl.ds(b * EB + j * 16, 16)]
                d16 = dst2_v[b, pl.ds(j * 16, 16)]
                ev = (plsc.load_gather(es_v, [s16])
                      + plsc.load_gather(ed_v, [d16]))
                ev = jnp.where(ev >= 0.0, ev, 0.2 * ev)
                ee_v[pl.ds(j * 16, 16)] = jnp.exp(ev)
            def _scale(e, _):
                e16 = jnp.full((16,), e, jnp.int32)
                sc = plsc.load_gather(ee_v, [e16])
                for k in range(HCOLS // 16):
                    o = k * 16
                    rows_v[ro + e, pl.ds(o, 16)] = (
                        rows_v[ro + e, pl.ds(o, 16)] * sc)
                return 0
            lax.fori_loop(0, EB, _scale, 0)
            _scatter(b, par).start(add=True)
            return 0
        lax.fori_loop(0, NBLK, _blk, 0)
        _scatter(NBLK - 1, (NBLK - 1) % 2).wait()
        plsc.subcore_barrier()

        rb = s * 624
        pltpu.sync_copy(num_s.at[pl.ds(rb, 624)],
                        nump.at[pl.ds(head * N_NODES + rb, 624)])
        @pl.when(s == NTILES - 1)
        def _tail():
            pltpu.sync_copy(num_s.at[pl.ds(9984, 16)],
                            nump.at[pl.ds(head * N_NODES + 9984, 16)])
        plsc.subcore_barrier()


_edge_kernel = pl.kernel(
    _edge_body,
    out_type=jax.ShapeDtypeStruct((NHEAD * N_NODES, HCOLS), jnp.float32),
    mesh=plsc.VectorSubcoreMesh(core_axis_name="c", subcore_axis_name="s"),
    compiler_params=pltpu.CompilerParams(
        needs_layout_passes=False, use_tc_tiling_on_sc=False),
    scratch_types=[
        pltpu.VMEM((EPT,), jnp.int32),
        pltpu.VMEM((EPT,), jnp.int32),
        pltpu.VMEM((NBLK, EB), jnp.int32),
        pltpu.VMEM((N_ES,), jnp.float32),
        pltpu.VMEM((N_ES,), jnp.float32),
        pltpu.VMEM((2 * EB, HCOLS), jnp.float32),
        pltpu.VMEM((EB,), jnp.float32),
        pltpu.VMEM_SHARED((N_ACC, HCOLS), jnp.float32),
        pltpu.SemaphoreType.DMA,
        pltpu.SemaphoreType.DMA,
    ],
)


def _ep_body(acc_ref, s_ref, b_ref, pr_ref, o_ref):
    parts = []
    for h in range(NHEAD):
        a = acc_ref[h]
        parts.append(a[:, :HD] / (a[:, HD:HD + 1] + 1e-9))
    out = jnp.concatenate(parts, axis=1)
    mu = jnp.mean(out, axis=1, keepdims=True)
    var = jnp.mean((out - mu) ** 2, axis=1, keepdims=True)
    y = (out - mu) / jnp.sqrt(var + 1e-5) * s_ref[...] + b_ref[...]
    o_ref[...] = jnp.where(y >= 0.0, y, pr_ref[0, 0] * y)


def _gat_epilogue(nump, ln_s, ln_b, prelu, bm=2000):
    return pl.pallas_call(
        _ep_body,
        grid=(N_NODES // bm,),
        in_specs=[
            pl.BlockSpec((NHEAD, bm, HCOLS), lambda i: (0, i, 0)),
            pl.BlockSpec((1, H), lambda i: (0, 0)),
            pl.BlockSpec((1, H), lambda i: (0, 0)),
            pl.BlockSpec((1, 1), lambda i: (0, 0)),
        ],
        out_specs=pl.BlockSpec((bm, H), lambda i: (i, 0)),
        out_shape=jax.ShapeDtypeStruct((N_NODES, H), jnp.float32),
    )(nump.reshape(NHEAD, N_NODES, HCOLS), ln_s.reshape(1, H),
      ln_b.reshape(1, H), prelu.reshape(1, 1))


def _gat_sc(h, p, src1p, dst2p):
    hw, es, ed = _hw_proj(h, p['W'], p['a_src'], p['a_dst'])
    pad = ((0, 0), (0, N_ES - N_NODES), (0, 0))
    esp = jnp.pad(es, pad, constant_values=NEG).reshape(NHEAD * N_ES)
    edp = jnp.pad(ed, pad, constant_values=NEG).reshape(NHEAD * N_ES)
    nump = _edge_kernel(
        hw.reshape(NHEAD * N_NODES, HCOLS), esp, edp, src1p, dst2p)
    return _gat_epilogue(nump, p['ln_s'], p['ln_b'], p['prelu'])


def _ln(x, scale=None, bias=None, eps=1e-5):
    mu = jnp.mean(x, axis=-1, keepdims=True)
    var = jnp.var(x, axis=-1, keepdims=True)
    y = (x - mu) / jnp.sqrt(var + eps)
    if scale is not None:
        y = y * scale + bias
    return y


def kernel(x, edge_index, t, noise_raw, params):
    src, dst = edge_index[0], edge_index[1]
    npad = NTILES * EPT - N_EDGES
    src1p = jnp.concatenate([src, jnp.zeros((npad,), jnp.int32)])
    dstf = jnp.concatenate([dst, jnp.full((npad,), DUMMY, jnp.int32)])
    dst2p = dstf.reshape(NTILES, NBLK, EB)

    betas = jnp.linspace(1e-4, 0.02, T_STEPS, dtype=jnp.float32)
    alphas_bar = jnp.cumprod(1.0 - betas)
    sab = jnp.sqrt(alphas_bar)
    somab = jnp.sqrt(1.0 - alphas_bar)

    layers = list(params['down']) + list(params['up'])
    tabs = [_mm_bias(params['time_embedding'], p['tW'], p['tb'])
            for p in layers]
    table = jnp.concatenate(
        tabs + [sab.reshape(T_STEPS, 1), somab.reshape(T_STEPS, 1),
                jnp.zeros((T_STEPS, 126), jnp.float32)], axis=1)
    g = _onehot_gather(t, table)
    temb_proj = [g[:, i * H:(i + 1) * H] for i in range(4)]
    sab_t = g[:, 4 * H:4 * H + 1]
    somab_t = g[:, 4 * H + 1:4 * H + 2]

    xn = _ln(x)
    miu = jnp.mean(xn, axis=0)
    std = jnp.std(xn, axis=0, ddof=1)
    noise = _ln(noise_raw)
    noise = noise * std + miu
    noise = jnp.sign(xn) * jnp.abs(noise)
    x_t = sab_t * xn + somab_t * noise

    h = _mm_bias(x_t, params['W_in'], params['b_in'])
    skips = []
    for i in range(NUM_LAYERS):
        p = params['down'][i]
        h = h + temb_proj[i]
        h = _gat_sc(h, p, src1p, dst2p)
        skips.append(h)
    for i in range(NUM_LAYERS):
        p = params['up'][i]
        h = h + skips[NUM_LAYERS - 1 - i]
        h = h + temb_proj[NUM_LAYERS + i]
        h = _gat_sc(h, p, src1p, dst2p)
    pred = _mm_bias(h, params['W_out'], params['b_out'])

    pn = pred / (jnp.linalg.norm(pred, axis=-1, keepdims=True) + 1e-12)
    tn = xn / (jnp.linalg.norm(xn, axis=-1, keepdims=True) + 1e-12)
    return jnp.mean((1.0 - jnp.sum(pn * tn, axis=-1)) ** ALPHA_L)


# speedup vs baseline: 1.0711x; 1.0110x over previous
import functools

import jax
import jax.numpy as jnp
from jax import lax
from jax.experimental import pallas as pl
from jax.experimental.pallas import tpu as pltpu
from jax.experimental.pallas import tpu_sc as plsc

N_NODES = 10000
N_EDGES = 160000
IN_DIM = 256
H = 512
NHEAD = 8
HD = H // NHEAD
NUM_LAYERS = 2
T_STEPS = 1000
ALPHA_L = 2.0

HCOLS = HD + 16
NTILES = 16
EPT = 10112
EB = 128
NBLK = EPT // EB
N_ES = 10008
N_ACC = 10112
DUMMY = N_NODES
NEG = -1.0e30


def _mm_bias_body(x_ref, w_ref, b_ref, o_ref):
    o_ref[...] = (
        jnp.dot(x_ref[...], w_ref[...], preferred_element_type=jnp.float32)
        + b_ref[...]
    )


def _mm_bias(x, w, b, bm=2000):
    m, k = x.shape
    n = w.shape[1]
    if m <= bm:
        bm = m
    assert m % bm == 0
    return pl.pallas_call(
        _mm_bias_body,
        grid=(m // bm,),
        in_specs=[
            pl.BlockSpec((bm, k), lambda i: (i, 0)),
            pl.BlockSpec((k, n), lambda i: (0, 0)),
            pl.BlockSpec((1, n), lambda i: (0, 0)),
        ],
        out_specs=pl.BlockSpec((bm, n), lambda i: (i, 0)),
        out_shape=jax.ShapeDtypeStruct((m, n), jnp.float32),
    )(x, w, b.reshape(1, n))


def _onehot_body(t_ref, tab_ref, o_ref):
    tb = t_ref[...]
    oh = (lax.broadcasted_iota(jnp.int32, (tb.shape[0], T_STEPS), 1)
          == tb).astype(jnp.float32)
    o_ref[...] = jnp.dot(oh, tab_ref[...], preferred_element_type=jnp.float32)


def _onehot_gather(t, table, bm=1000):
    m = t.shape[0]
    n = table.shape[1]
    return pl.pallas_call(
        _onehot_body,
        grid=(m // bm,),
        in_specs=[
            pl.BlockSpec((bm, 1), lambda i: (i, 0)),
            pl.BlockSpec((T_STEPS, n), lambda i: (0, 0)),
        ],
        out_specs=pl.BlockSpec((bm, n), lambda i: (i, 0)),
        out_shape=jax.ShapeDtypeStruct((m, n), jnp.float32),
    )(t.reshape(m, 1), table)


def _hw_body(h_ref, a1_ref, w_ref, asrc_ref, adst_ref, hw_ref, es_ref, ed_ref):
    hin = h_ref[...] + a1_ref[...]
    blk = jnp.dot(hin, w_ref[0], preferred_element_type=jnp.float32)
    bm = blk.shape[0]
    ii = lax.broadcasted_iota(jnp.int32, (bm, 16), 1)
    ones_pat = jnp.where(ii == 0, 1.0, 0.0).astype(jnp.float32)
    hw_ref[0] = jnp.concatenate([blk, ones_pat], axis=1)
    es_ref[0] = jnp.sum(blk * asrc_ref[0], axis=1, keepdims=True)
    ed_ref[0] = jnp.sum(blk * adst_ref[0], axis=1, keepdims=True)


def _hw_proj(h, add1, W, a_src, a_dst, bm=2000):
    m = h.shape[0]
    grid = (m // bm, NHEAD)
    hw, es, ed = pl.pallas_call(
        _hw_body,
        grid=grid,
        in_specs=[
            pl.BlockSpec((bm, H), lambda i, j: (i, 0)),
            pl.BlockSpec((bm, H), lambda i, j: (i, 0)),
            pl.BlockSpec((1, H, HD), lambda i, j: (j, 0, 0)),
            pl.BlockSpec((1, 1, HD), lambda i, j: (j, 0, 0)),
            pl.BlockSpec((1, 1, HD), lambda i, j: (j, 0, 0)),
        ],
        out_specs=[
            pl.BlockSpec((1, bm, HCOLS), lambda i, j: (j, i, 0)),
            pl.BlockSpec((1, bm, 1), lambda i, j: (j, i, 0)),
            pl.BlockSpec((1, bm, 1), lambda i, j: (j, i, 0)),
        ],
        out_shape=[
            jax.ShapeDtypeStruct((NHEAD, m, HCOLS), jnp.float32),
            jax.ShapeDtypeStruct((NHEAD, m, 1), jnp.float32),
            jax.ShapeDtypeStruct((NHEAD, m, 1), jnp.float32),
        ],
    )(h, add1, W.reshape(H, NHEAD, HD).transpose(1, 0, 2),
      a_src.reshape(NHEAD, 1, HD), a_dst.reshape(NHEAD, 1, HD))
    return hw, es, ed


def _edge_body(hwp, esp, edp, src1p, dst2p, nump,
               src1_v, srcoff_v, dst2_v, es_v, ed_v,
               rows_v, ee_v, num_s, sem, sem_sc):
    c = lax.axis_index("c")
    s = lax.axis_index("s")
    zf16 = jnp.zeros((16,), jnp.float32)

    pltpu.sync_copy(src1p.at[pl.ds(s * EPT, EPT)], src1_v)
    pltpu.sync_copy(dst2p.at[s], dst2_v)

    for phase in range(4):
        head = c * 4 + phase

        pltpu.sync_copy(esp.at[pl.ds(head * N_ES, N_ES)], es_v)
        pltpu.sync_copy(edp.at[pl.ds(head * N_ES, N_ES)], ed_v)

        def _off(k, _):
            srcoff_v[pl.ds(k * 16, 16)] = (
                src1_v[pl.ds(k * 16, 16)] + head * N_NODES)
            return 0
        lax.fori_loop(0, EPT // 16, _off, 0)

        def _zr(r, _):
            for j in range(HCOLS // 16):
                rows_v[r, pl.ds(j * 16, 16)] = zf16
            return 0
        lax.fori_loop(0, EB, _zr, 0)
        base = s * (N_ACC // NTILES)
        for k in range(4):
            pltpu.sync_copy(rows_v.at[pl.ds(0, EB)], num_s.at[pl.ds(base + k * EB, EB)])
        pltpu.sync_copy(rows_v.at[pl.ds(0, 120)],
                        num_s.at[pl.ds(base + 4 * EB, 120)])
        plsc.subcore_barrier()

        def _gather(b, par):
            return pltpu.make_async_copy(
                hwp.at[srcoff_v.at[pl.ds(b * EB, EB)]],
                rows_v.at[pl.ds(par * EB, EB)], sem)

        def _scatter(b, par):
            return pltpu.make_async_copy(
                rows_v.at[pl.ds(par * EB, EB)],
                num_s.at[dst2_v.at[b]], sem_sc)

        _gather(0, 0).start()
        def _blk(b, _):
            par = lax.rem(b, 2)
            @pl.when(b >= 1)
            def _drain():
                _scatter(b - 1, 1 - par).wait()
            @pl.when(b < NBLK - 1)
            def _next():
                _gather(b + 1, 1 - par).start()
            _gather(b, par).wait()
            ro = par * EB
            for j in range(8):
                s16 = src1_v[pl.ds(b * EB + j * 16, 16)]
                d16 = dst2_v[b, pl.ds(j * 16, 16)]
                ev = (plsc.load_gather(es_v, [s16])
                      + plsc.load_gather(ed_v, [d16]))
                ev = jnp.where(ev >= 0.0, ev, 0.2 * ev)
                ee_v[pl.ds(j * 16, 16)] = jnp.exp(ev)
            def _scale(e, _):
                e16 = jnp.full((16,), e, jnp.int32)
                sc = plsc.load_gather(ee_v, [e16])
                for k in range(HCOLS // 16):
                    o = k * 16
                    rows_v[ro + e, pl.ds(o, 16)] = (
                        rows_v[ro + e, pl.ds(o, 16)] * sc)
                return 0
            lax.fori_loop(0, EB, _scale, 0)
            _scatter(b, par).start(add=True)
            return 0
        lax.fori_loop(0, NBLK, _blk, 0)
        _scatter(NBLK - 1, (NBLK - 1) % 2).wait()
        plsc.subcore_barrier()

        rb = s * 624
        pltpu.sync_copy(num_s.at[pl.ds(rb, 624)],
                        nump.at[pl.ds(head * N_NODES + rb, 624)])
        @pl.when(s == NTILES - 1)
        def _tail():
            pltpu.sync_copy(num_s.at[pl.ds(9984, 16)],
                            nump.at[pl.ds(head * N_NODES + 9984, 16)])
        plsc.subcore_barrier()


_edge_kernel = pl.kernel(
    _edge_body,
    out_type=jax.ShapeDtypeStruct((NHEAD * N_NODES, HCOLS), jnp.float32),
    mesh=plsc.VectorSubcoreMesh(core_axis_name="c", subcore_axis_name="s"),
    compiler_params=pltpu.CompilerParams(
        needs_layout_passes=False, use_tc_tiling_on_sc=False),
    scratch_types=[
        pltpu.VMEM((EPT,), jnp.int32),
        pltpu.VMEM((EPT,), jnp.int32),
        pltpu.VMEM((NBLK, EB), jnp.int32),
        pltpu.VMEM((N_ES,), jnp.float32),
        pltpu.VMEM((N_ES,), jnp.float32),
        pltpu.VMEM((2 * EB, HCOLS), jnp.float32),
        pltpu.VMEM((EB,), jnp.float32),
        pltpu.VMEM_SHARED((N_ACC, HCOLS), jnp.float32),
        pltpu.SemaphoreType.DMA,
        pltpu.SemaphoreType.DMA,
    ],
)


def _ep_body(acc_ref, s_ref, b_ref, pr_ref, o_ref):
    parts = []
    for h in range(NHEAD):
        a = acc_ref[h]
        parts.append(a[:, :HD] / (a[:, HD:HD + 1] + 1e-9))
    out = jnp.concatenate(parts, axis=1)
    mu = jnp.mean(out, axis=1, keepdims=True)
    var = jnp.mean((out - mu) ** 2, axis=1, keepdims=True)
    y = (out - mu) / jnp.sqrt(var + 1e-5) * s_ref[...] + b_ref[...]
    o_ref[...] = jnp.where(y >= 0.0, y, pr_ref[0, 0] * y)


def _gat_epilogue(nump, ln_s, ln_b, prelu, bm=2000):
    return pl.pallas_call(
        _ep_body,
        grid=(N_NODES // bm,),
        in_specs=[
            pl.BlockSpec((NHEAD, bm, HCOLS), lambda i: (0, i, 0)),
            pl.BlockSpec((1, H), lambda i: (0, 0)),
            pl.BlockSpec((1, H), lambda i: (0, 0)),
            pl.BlockSpec((1, 1), lambda i: (0, 0)),
        ],
        out_specs=pl.BlockSpec((bm, H), lambda i: (i, 0)),
        out_shape=jax.ShapeDtypeStruct((N_NODES, H), jnp.float32),
    )(nump.reshape(NHEAD, N_NODES, HCOLS), ln_s.reshape(1, H),
      ln_b.reshape(1, H), prelu.reshape(1, 1))


def _gat_sc(h, add1, p, src1p, dst2p):
    hw, es, ed = _hw_proj(h, add1, p['W'], p['a_src'], p['a_dst'])
    pad = ((0, 0), (0, N_ES - N_NODES), (0, 0))
    esp = jnp.pad(es, pad, constant_values=NEG).reshape(NHEAD * N_ES)
    edp = jnp.pad(ed, pad, constant_values=NEG).reshape(NHEAD * N_ES)
    nump = _edge_kernel(
        hw.reshape(NHEAD * N_NODES, HCOLS), esp, edp, src1p, dst2p)
    return _gat_epilogue(nump, p['ln_s'], p['ln_b'], p['prelu'])


def _ln(x, scale=None, bias=None, eps=1e-5):
    mu = jnp.mean(x, axis=-1, keepdims=True)
    var = jnp.var(x, axis=-1, keepdims=True)
    y = (x - mu) / jnp.sqrt(var + eps)
    if scale is not None:
        y = y * scale + bias
    return y


def kernel(x, edge_index, t, noise_raw, params):
    src, dst = edge_index[0], edge_index[1]
    npad = NTILES * EPT - N_EDGES
    src1p = jnp.concatenate([src, jnp.zeros((npad,), jnp.int32)])
    dstf = jnp.concatenate([dst, jnp.full((npad,), DUMMY, jnp.int32)])
    dst2p = dstf.reshape(NTILES, NBLK, EB)

    betas = jnp.linspace(1e-4, 0.02, T_STEPS, dtype=jnp.float32)
    alphas_bar = jnp.cumprod(1.0 - betas)
    sab = jnp.sqrt(alphas_bar)
    somab = jnp.sqrt(1.0 - alphas_bar)

    layers = list(params['down']) + list(params['up'])
    tabs = [_mm_bias(params['time_embedding'], p['tW'], p['tb'])
            for p in layers]
    table = jnp.concatenate(
        tabs + [sab.reshape(T_STEPS, 1), somab.reshape(T_STEPS, 1),
                jnp.zeros((T_STEPS, 126), jnp.float32)], axis=1)
    g = _onehot_gather(t, table)
    temb_proj = [g[:, i * H:(i + 1) * H] for i in range(4)]
    sab_t = g[:, 4 * H:4 * H + 1]
    somab_t = g[:, 4 * H + 1:4 * H + 2]

    xn = _ln(x)
    miu = jnp.mean(xn, axis=0)
    std = jnp.std(xn, axis=0, ddof=1)
    noise = _ln(noise_raw)
    noise = noise * std + miu
    noise = jnp.sign(xn) * jnp.abs(noise)
    x_t = sab_t * xn + somab_t * noise

    h = _mm_bias(x_t, params['W_in'], params['b_in'])
    skips = []
    for i in range(NUM_LAYERS):
        p = params['down'][i]
        h = _gat_sc(h, temb_proj[i], p, src1p, dst2p)
        skips.append(h)
    for i in range(NUM_LAYERS):
        p = params['up'][i]
        add1 = skips[NUM_LAYERS - 1 - i] + temb_proj[NUM_LAYERS + i]
        h = _gat_sc(h, add1, p, src1p, dst2p)
    pred = _mm_bias(h, params['W_out'], params['b_out'])

    pn = pred / (jnp.linalg.norm(pred, axis=-1, keepdims=True) + 1e-12)
    tn = xn / (jnp.linalg.norm(xn, axis=-1, keepdims=True) + 1e-12)
    return jnp.mean((1.0 - jnp.sum(pn * tn, axis=-1)) ** ALPHA_L)


# speedup vs baseline: 1.0713x; 1.0002x over previous
import jax
import jax.numpy as jnp
from jax import lax
from jax.experimental import pallas as pl
from jax.experimental.pallas import tpu as pltpu
from jax.experimental.pallas import tpu_sc as plsc

N_NODES = 10000
N_EDGES = 160000
IN_DIM = 256
H = 512
NHEAD = 8
HD = H // NHEAD
NUM_LAYERS = 2
T_STEPS = 1000
ALPHA_L = 2.0

HCOLS = HD + 16
NTILES = 16
EPT = 10112
EB = 128
NBLK = EPT // EB
N_ES = 10008
N_ACC = 10112
DUMMY = N_NODES
NEG = -1.0e30


def _mm_bias_body(x_ref, w_ref, b_ref, o_ref):
    o_ref[...] = (
        jnp.dot(x_ref[...], w_ref[...], preferred_element_type=jnp.float32)
        + b_ref[...]
    )


def _mm_bias(x, w, b, bm=2000):
    m, k = x.shape
    n = w.shape[1]
    if m <= bm:
        bm = m
    assert m % bm == 0
    return pl.pallas_call(
        _mm_bias_body,
        grid=(m // bm,),
        in_specs=[
            pl.BlockSpec((bm, k), lambda i: (i, 0)),
            pl.BlockSpec((k, n), lambda i: (0, 0)),
            pl.BlockSpec((1, n), lambda i: (0, 0)),
        ],
        out_specs=pl.BlockSpec((bm, n), lambda i: (i, 0)),
        out_shape=jax.ShapeDtypeStruct((m, n), jnp.float32),
    )(x, w, b.reshape(1, n))


def _onehot_body(t_ref, tab_ref, o_ref):
    tb = t_ref[...]
    oh = (lax.broadcasted_iota(jnp.int32, (tb.shape[0], T_STEPS), 1)
          == tb).astype(jnp.float32)
    o_ref[...] = jnp.dot(oh, tab_ref[...], preferred_element_type=jnp.float32)


def _onehot_gather(t, table, bm=1000):
    m = t.shape[0]
    n = table.shape[1]
    return pl.pallas_call(
        _onehot_body,
        grid=(m // bm,),
        in_specs=[
            pl.BlockSpec((bm, 1), lambda i: (i, 0)),
            pl.BlockSpec((T_STEPS, n), lambda i: (0, 0)),
        ],
        out_specs=pl.BlockSpec((bm, n), lambda i: (i, 0)),
        out_shape=jax.ShapeDtypeStruct((m, n), jnp.float32),
    )(t.reshape(m, 1), table)


def _hw_body(h_ref, a1_ref, w_ref, asrc_ref, adst_ref, hw_ref, es_ref, ed_ref):
    hin = h_ref[...] + a1_ref[...]
    blk = jnp.dot(hin, w_ref[0], preferred_element_type=jnp.float32)
    bm = blk.shape[0]
    ii = lax.broadcasted_iota(jnp.int32, (bm, 16), 1)
    ones_pat = jnp.where(ii == 0, 1.0, 0.0).astype(jnp.float32)
    hw_ref[0] = jnp.concatenate([blk, ones_pat], axis=1)
    es_ref[0] = jnp.sum(blk * asrc_ref[0], axis=1, keepdims=True)
    ed_ref[0] = jnp.sum(blk * adst_ref[0], axis=1, keepdims=True)


def _hw_proj(h, add1, W, a_src, a_dst, bm=2000):
    m = h.shape[0]
    grid = (m // bm, NHEAD)
    hw, es, ed = pl.pallas_call(
        _hw_body,
        grid=grid,
        in_specs=[
            pl.BlockSpec((bm, H), lambda i, j: (i, 0)),
            pl.BlockSpec((bm, H), lambda i, j: (i, 0)),
            pl.BlockSpec((1, H, HD), lambda i, j: (j, 0, 0)),
            pl.BlockSpec((1, 1, HD), lambda i, j: (j, 0, 0)),
            pl.BlockSpec((1, 1, HD), lambda i, j: (j, 0, 0)),
        ],
        out_specs=[
            pl.BlockSpec((1, bm, HCOLS), lambda i, j: (j, i, 0)),
            pl.BlockSpec((1, bm, 1), lambda i, j: (j, i, 0)),
            pl.BlockSpec((1, bm, 1), lambda i, j: (j, i, 0)),
        ],
        out_shape=[
            jax.ShapeDtypeStruct((NHEAD, m, HCOLS), jnp.float32),
            jax.ShapeDtypeStruct((NHEAD, m, 1), jnp.float32),
            jax.ShapeDtypeStruct((NHEAD, m, 1), jnp.float32),
        ],
    )(h, add1, W.reshape(H, NHEAD, HD).transpose(1, 0, 2),
      a_src.reshape(NHEAD, 1, HD), a_dst.reshape(NHEAD, 1, HD))
    return hw, es, ed


def _edge_body(hwp, esp, edp, src1p, dst2p, nump,
               src1_v, srcoff_v, dst2_v, es_v, ed_v,
               rows_v, ee_v, num_s, sem, sem_sc):
    c = lax.axis_index("c")
    s = lax.axis_index("s")
    zf16 = jnp.zeros((16,), jnp.float32)

    pltpu.sync_copy(src1p.at[pl.ds(s * EPT, EPT)], src1_v)
    pltpu.sync_copy(dst2p.at[s], dst2_v)

    for phase in range(4):
        head = c * 4 + phase

        pltpu.sync_copy(esp.at[pl.ds(head * N_ES, N_ES)], es_v)
        pltpu.sync_copy(edp.at[pl.ds(head * N_ES, N_ES)], ed_v)

        def _off(k, _):
            srcoff_v[pl.ds(k * 16, 16)] = (
                src1_v[pl.ds(k * 16, 16)] + head * N_NODES)
            return 0
        lax.fori_loop(0, EPT // 16, _off, 0)

        def _zr(r, _):
            for j in range(HCOLS // 16):
                rows_v[r, pl.ds(j * 16, 16)] = zf16
            return 0
        lax.fori_loop(0, EB, _zr, 0)
        base = s * (N_ACC // NTILES)
        for k in range(4):
            pltpu.sync_copy(rows_v.at[pl.ds(0, EB)], num_s.at[pl.ds(base + k * EB, EB)])
        pltpu.sync_copy(rows_v.at[pl.ds(0, 120)],
                        num_s.at[pl.ds(base + 4 * EB, 120)])
        plsc.subcore_barrier()

        def _gather(b, par):
            return pltpu.make_async_copy(
                hwp.at[srcoff_v.at[pl.ds(b * EB, EB)]],
                rows_v.at[pl.ds(par * EB, EB)], sem)

        def _scatter(b, par):
            return pltpu.make_async_copy(
                rows_v.at[pl.ds(par * EB, EB)],
                num_s.at[dst2_v.at[b]], sem_sc)

        _gather(0, 0).start()
        def _blk(b, _):
            par = lax.rem(b, 2)
            @pl.when(b >= 1)
            def _drain():
                _scatter(b - 1, 1 - par).wait()
            @pl.when(b < NBLK - 1)
            def _next():
                _gather(b + 1, 1 - par).start()
            _gather(b, par).wait()
            ro = par * EB
            for j in range(8):
                s16 = src1_v[pl.ds(b * EB + j * 16, 16)]
                d16 = dst2_v[b, pl.ds(j * 16, 16)]
                ev = (plsc.load_gather(es_v, [s16])
                      + plsc.load_gather(ed_v, [d16]))
                ev = jnp.where(ev >= 0.0, ev, 0.2 * ev)
                ee_v[pl.ds(j * 16, 16)] = jnp.exp(ev)
            def _scale(e, _):
                e16 = jnp.full((16,), e, jnp.int32)
                sc = plsc.load_gather(ee_v, [e16])
                for k in range(HCOLS // 16):
                    o = k * 16
                    rows_v[ro + e, pl.ds(o, 16)] = (
                        rows_v[ro + e, pl.ds(o, 16)] * sc)
                return 0
            lax.fori_loop(0, EB, _scale, 0)
            _scatter(b, par).start(add=True)
            return 0
        lax.fori_loop(0, NBLK, _blk, 0)
        _scatter(NBLK - 1, (NBLK - 1) % 2).wait()
        plsc.subcore_barrier()

        rb = s * 624
        pltpu.sync_copy(num_s.at[pl.ds(rb, 624)],
                        nump.at[pl.ds(head * N_NODES + rb, 624)])
        @pl.when(s == NTILES - 1)
        def _tail():
            pltpu.sync_copy(num_s.at[pl.ds(9984, 16)],
                            nump.at[pl.ds(head * N_NODES + 9984, 16)])
        plsc.subcore_barrier()


_edge_kernel = pl.kernel(
    _edge_body,
    out_type=jax.ShapeDtypeStruct((NHEAD * N_NODES, HCOLS), jnp.float32),
    mesh=plsc.VectorSubcoreMesh(core_axis_name="c", subcore_axis_name="s"),
    compiler_params=pltpu.CompilerParams(
        needs_layout_passes=False, use_tc_tiling_on_sc=False),
    scratch_types=[
        pltpu.VMEM((EPT,), jnp.int32),
        pltpu.VMEM((EPT,), jnp.int32),
        pltpu.VMEM((NBLK, EB), jnp.int32),
        pltpu.VMEM((N_ES,), jnp.float32),
        pltpu.VMEM((N_ES,), jnp.float32),
        pltpu.VMEM((2 * EB, HCOLS), jnp.float32),
        pltpu.VMEM((EB,), jnp.float32),
        pltpu.VMEM_SHARED((N_ACC, HCOLS), jnp.float32),
        pltpu.SemaphoreType.DMA,
        pltpu.SemaphoreType.DMA,
    ],
)


def _ep_body(acc_ref, s_ref, b_ref, pr_ref, o_ref):
    parts = []
    for h in range(NHEAD):
        a = acc_ref[h]
        parts.append(a[:, :HD] / (a[:, HD:HD + 1] + 1e-9))
    out = jnp.concatenate(parts, axis=1)
    mu = jnp.mean(out, axis=1, keepdims=True)
    var = jnp.mean((out - mu) ** 2, axis=1, keepdims=True)
    y = (out - mu) / jnp.sqrt(var + 1e-5) * s_ref[...] + b_ref[...]
    o_ref[...] = jnp.where(y >= 0.0, y, pr_ref[0, 0] * y)


def _gat_epilogue(nump, ln_s, ln_b, prelu, bm=2000):
    return pl.pallas_call(
        _ep_body,
        grid=(N_NODES // bm,),
        in_specs=[
            pl.BlockSpec((NHEAD, bm, HCOLS), lambda i: (0, i, 0)),
            pl.BlockSpec((1, H), lambda i: (0, 0)),
            pl.BlockSpec((1, H), lambda i: (0, 0)),
            pl.BlockSpec((1, 1), lambda i: (0, 0)),
        ],
        out_specs=pl.BlockSpec((bm, H), lambda i: (i, 0)),
        out_shape=jax.ShapeDtypeStruct((N_NODES, H), jnp.float32),
    )(nump.reshape(NHEAD, N_NODES, HCOLS), ln_s.reshape(1, H),
      ln_b.reshape(1, H), prelu.reshape(1, 1))


def _gat_sc(h, add1, p, src1p, dst2p):
    hw, es, ed = _hw_proj(h, add1, p['W'], p['a_src'], p['a_dst'])
    pad = ((0, 0), (0, N_ES - N_NODES), (0, 0))
    esp = jnp.pad(es, pad, constant_values=NEG).reshape(NHEAD * N_ES)
    edp = jnp.pad(ed, pad, constant_values=NEG).reshape(NHEAD * N_ES)
    nump = _edge_kernel(
        hw.reshape(NHEAD * N_NODES, HCOLS), esp, edp, src1p, dst2p)
    return _gat_epilogue(nump, p['ln_s'], p['ln_b'], p['prelu'])


def _ln(x, scale=None, bias=None, eps=1e-5):
    mu = jnp.mean(x, axis=-1, keepdims=True)
    var = jnp.var(x, axis=-1, keepdims=True)
    y = (x - mu) / jnp.sqrt(var + eps)
    if scale is not None:
        y = y * scale + bias
    return y


def kernel(x, edge_index, t, noise_raw, params):
    src, dst = edge_index[0], edge_index[1]
    npad = NTILES * EPT - N_EDGES
    src1p = jnp.concatenate([src, jnp.zeros((npad,), jnp.int32)])
    dstf = jnp.concatenate([dst, jnp.full((npad,), DUMMY, jnp.int32)])
    dst2p = dstf.reshape(NTILES, NBLK, EB)

    betas = jnp.linspace(1e-4, 0.02, T_STEPS, dtype=jnp.float32)
    alphas_bar = jnp.cumprod(1.0 - betas)
    sab = jnp.sqrt(alphas_bar)
    somab = jnp.sqrt(1.0 - alphas_bar)

    layers = list(params['down']) + list(params['up'])
    tabs = [_mm_bias(params['time_embedding'], p['tW'], p['tb'])
            for p in layers]
    table = jnp.concatenate(
        tabs + [sab.reshape(T_STEPS, 1), somab.reshape(T_STEPS, 1),
                jnp.zeros((T_STEPS, 126), jnp.float32)], axis=1)
    g = _onehot_gather(t, table)
    temb_proj = [g[:, i * H:(i + 1) * H] for i in range(4)]
    sab_t = g[:, 4 * H:4 * H + 1]
    somab_t = g[:, 4 * H + 1:4 * H + 2]

    xn = _ln(x)
    miu = jnp.mean(xn, axis=0)
    std = jnp.std(xn, axis=0, ddof=1)
    noise = _ln(noise_raw)
    noise = noise * std + miu
    noise = jnp.sign(xn) * jnp.abs(noise)
    x_t = sab_t * xn + somab_t * noise

    h = _mm_bias(x_t, params['W_in'], params['b_in'])
    skips = []
    for i in range(NUM_LAYERS):
        p = params['down'][i]
        h = _gat_sc(h, temb_proj[i], p, src1p, dst2p)
        skips.append(h)
    for i in range(NUM_LAYERS):
        p = params['up'][i]
        add1 = skips[NUM_LAYERS - 1 - i] + temb_proj[NUM_LAYERS + i]
        h = _gat_sc(h, add1, p, src1p, dst2p)
    pred = _mm_bias(h, params['W_out'], params['b_out'])

    pn = pred / (jnp.linalg.norm(pred, axis=-1, keepdims=True) + 1e-12)
    tn = xn / (jnp.linalg.norm(xn, axis=-1, keepdims=True) + 1e-12)
    return jnp.mean((1.0 - jnp.sum(pn * tn, axis=-1)) ** ALPHA_L)
